# baseline (speedup 1.0000x reference)
import functools

import jax
import jax.numpy as jnp
from jax import lax
from jax.experimental import pallas as pl
from jax.experimental.pallas import tpu as pltpu
from jax.experimental.pallas import tpu_sc as plsc

N = 10000
E = 320000
D = 128
NG = 64
NCLS = 10

NCORES = 2
NSUB = 16
NW = NCORES * NSUB
EPT = E // NW
CH = 32
CPT = 313
EPTP = CH * CPT
DUMP = 16
NDCH = N // DUMP
DCPT = -(-NDCH // NSUB)

def _zero_acc(page_v, acc_sh, sid):
    zero16 = jnp.zeros((16,), jnp.float32)

    @pl.loop(0, DUMP)
    def _(i):
        for j in range(D // 16):
            page_v[i, pl.ds(j * 16, 16)] = zero16

    @pl.loop(0, DCPT)
    def _(k):
        j = k * NSUB + sid

        @pl.when(j < NDCH)
        def _():
            pltpu.sync_copy(page_v, acc_sh.at[pl.ds(j * DUMP, DUMP)])

    plsc.subcore_barrier()


def _dump_acc(page_v, acc_sh, out_hbm, cid, sid):
    plsc.subcore_barrier()

    @pl.loop(0, DCPT)
    def _(k):
        j = k * NSUB + sid

        @pl.when(j < NDCH)
        def _():
            pltpu.sync_copy(acc_sh.at[pl.ds(j * DUMP, DUMP)], page_v)
            pltpu.sync_copy(page_v, out_hbm.at[cid, pl.ds(j * DUMP, DUMP)])


def _edge_body(y_hbm, idx_hbm, w_hbm, out_hbm,
               idx_v, wc_v, rows_v, acc_sh, sem):
    cid = lax.axis_index("c")
    sid = lax.axis_index("s")
    wid = sid * NCORES + cid

    _zero_acc(rows_v.at[pl.ds(0, DUMP)], acc_sh, sid)

    gdn = lax.GatherDimensionNumbers(
        offset_dims=(), collapsed_slice_dims=(0,), start_index_map=(0,))

    @pl.loop(0, CPT)
    def _(c):
        pltpu.sync_copy(idx_hbm.at[wid, c], idx_v)
        pltpu.sync_copy(w_hbm.at[wid, c], wc_v)
        pltpu.async_copy(y_hbm.at[idx_v.at[0]], rows_v, sem).wait()

        for g in range(CH // 16):
            wv = wc_v[0, pl.ds(g * 16, 16)]

            @pl.loop(0, 16, unroll=4)
            def _(r2):
                idxv = jnp.full((16, 1), r2, jnp.int32)
                wbc = lax.gather(wv, idxv, gdn, (1,),
                                 mode=lax.GatherScatterMode.PROMISE_IN_BOUNDS)
                rr = g * 16 + r2
                for j in range(D // 16):
                    rows_v[rr, pl.ds(j * 16, 16)] = (
                        rows_v[rr, pl.ds(j * 16, 16)] * wbc)

        pltpu.sync_copy(rows_v, acc_sh.at[idx_v.at[1]], add=True)

    _dump_acc(rows_v.at[pl.ds(0, DUMP)], acc_sh, out_hbm, cid, sid)


@functools.lru_cache(maxsize=None)
def _sc_kernels():
    mesh = plsc.VectorSubcoreMesh(core_axis_name="c", subcore_axis_name="s")
    params = pltpu.CompilerParams(needs_layout_passes=False)
    edge = pl.kernel(
        _edge_body,
        out_type=jax.ShapeDtypeStruct((NCORES, N, D), jnp.float32),
        mesh=mesh,
        compiler_params=params,
        scratch_types=[
            pltpu.VMEM((2, CH), jnp.int32),
            pltpu.VMEM((1, CH), jnp.float32),
            pltpu.VMEM((CH, D), jnp.float32),
            pltpu.VMEM_SHARED((N, D), jnp.float32),
            pltpu.SemaphoreType.DMA,
        ],
    )
    return (edge,)


def _edge_kernel(y, idx4, w4):
    return _sc_kernels()[0](y, idx4, w4)


RB = 1000
GRID = N // RB


def _tc1_body(x_ref, win_ref, bin_ref, w1_ref, dg0_ref, dg1_ref, y_ref, dis_ref):
    deg = dg0_ref[...] + dg1_ref[...] + 1.0
    dis = lax.rsqrt(deg)
    dis_ref[...] = dis
    h = jnp.maximum(jnp.dot(x_ref[...], win_ref[...],
                            preferred_element_type=jnp.float32) + bin_ref[...], 0.0)
    y_ref[...] = dis * jnp.dot(h, w1_ref[...], preferred_element_type=jnp.float32)


def _tc1(x, W_in, b_in, W1, dg0, dg1):
    return pl.pallas_call(
        _tc1_body,
        grid=(GRID,),
        in_specs=[
            pl.BlockSpec((RB, D), lambda i: (i, 0)),
            pl.BlockSpec((D, D), lambda i: (0, 0)),
            pl.BlockSpec((1, D), lambda i: (0, 0)),
            pl.BlockSpec((D, D), lambda i: (0, 0)),
            pl.BlockSpec((RB, 1), lambda i: (i, 0)),
            pl.BlockSpec((RB, 1), lambda i: (i, 0)),
        ],
        out_specs=[
            pl.BlockSpec((RB, D), lambda i: (i, 0)),
            pl.BlockSpec((RB, 1), lambda i: (i, 0)),
        ],
        out_shape=[
            jax.ShapeDtypeStruct((N, D), jnp.float32),
            jax.ShapeDtypeStruct((N, 1), jnp.float32),
        ],
    )(x, W_in, b_in, W1, dg0, dg1)


def _tc2_body(p0_ref, p1_ref, y_ref, dis_ref, b_ref, w_ref, out_ref):
    dis = dis_ref[...]
    h = jnp.maximum(dis * (p0_ref[...] + p1_ref[...] + y_ref[...]) + b_ref[...], 0.0)
    out_ref[...] = dis * jnp.dot(h, w_ref[...], preferred_element_type=jnp.float32)


def _tc2(p0, p1, y, dis, b, W):
    return pl.pallas_call(
        _tc2_body,
        grid=(GRID,),
        in_specs=[
            pl.BlockSpec((RB, D), lambda i: (i, 0)),
            pl.BlockSpec((RB, D), lambda i: (i, 0)),
            pl.BlockSpec((RB, D), lambda i: (i, 0)),
            pl.BlockSpec((RB, 1), lambda i: (i, 0)),
            pl.BlockSpec((1, D), lambda i: (0, 0)),
            pl.BlockSpec((D, D), lambda i: (0, 0)),
        ],
        out_specs=pl.BlockSpec((RB, D), lambda i: (i, 0)),
        out_shape=jax.ShapeDtypeStruct((N, D), jnp.float32),
    )(p0, p1, y, dis, b, W)


def _tc3_body(p0_ref, p1_ref, y_ref, dis_ref, b_ref, batch_ref, wc_ref, bc_ref,
              out_ref, sums_ref, cnts_ref):
    i = pl.program_id(0)

    @pl.when(i == 0)
    def _():
        sums_ref[...] = jnp.zeros_like(sums_ref)
        cnts_ref[...] = jnp.zeros_like(cnts_ref)

    dis = dis_ref[...]
    h = jnp.maximum(dis * (p0_ref[...] + p1_ref[...] + y_ref[...]) + b_ref[...], 0.0)
    b = batch_ref[...]
    iota = lax.broadcasted_iota(jnp.int32, (RB, NG), 1)
    onehot = (iota == b).astype(jnp.float32)
    dn = (((0,), (0,)), ((), ()))
    sums_ref[...] += lax.dot_general(onehot, h, dn,
                                     preferred_element_type=jnp.float32)
    cnts_ref[...] += lax.dot_general(onehot, jnp.ones((RB, 1), jnp.float32), dn,
                                     preferred_element_type=jnp.float32)

    @pl.when(i == GRID - 1)
    def _():
        rep = sums_ref[...] / jnp.maximum(cnts_ref[...], 1.0)
        out_ref[...] = jnp.dot(rep, wc_ref[...],
                               preferred_element_type=jnp.float32) + bc_ref[...]


def _tc3(p0, p1, y, dis, b, batch2, Wc, bc):
    return pl.pallas_call(
        _tc3_body,
        grid=(GRID,),
        in_specs=[
            pl.BlockSpec((RB, D), lambda i: (i, 0)),
            pl.BlockSpec((RB, D), lambda i: (i, 0)),
            pl.BlockSpec((RB, D), lambda i: (i, 0)),
            pl.BlockSpec((RB, 1), lambda i: (i, 0)),
            pl.BlockSpec((1, D), lambda i: (0, 0)),
            pl.BlockSpec((RB, 1), lambda i: (i, 0)),
            pl.BlockSpec((D, NCLS), lambda i: (0, 0)),
            pl.BlockSpec((1, NCLS), lambda i: (0, 0)),
        ],
        out_specs=pl.BlockSpec((NG, NCLS), lambda i: (0, 0)),
        out_shape=jax.ShapeDtypeStruct((NG, NCLS), jnp.float32),
        scratch_shapes=[
            pltpu.VMEM((NG, D), jnp.float32),
            pltpu.VMEM((NG, 1), jnp.float32),
        ],
    )(p0, p1, y, dis, b, batch2, Wc, bc)


def kernel(x, edge_index, edge_weights, batch, W_in, b_in, W1, b1, W2, b2, Wc, bc):
    src = edge_index[0].astype(jnp.int32)
    dst = edge_index[1].astype(jnp.int32)
    pad = ((0, 0), (0, EPTP - EPT))
    src3 = jnp.pad(src.reshape(NW, EPT), pad).reshape(NW, CPT, CH)
    dst3 = jnp.pad(dst.reshape(NW, EPT), pad).reshape(NW, CPT, CH)
    idx4 = jnp.stack([src3, dst3], axis=2)
    w4 = jnp.pad(edge_weights.astype(jnp.float32).reshape(NW, EPT),
                 pad).reshape(NW, CPT, 1, CH)

    ones_t = jnp.ones((N, D), jnp.float32)
    deg_parts = _edge_kernel(ones_t, idx4, w4)
    dg0 = lax.slice(deg_parts, (0, 0, 0), (1, N, 1)).reshape(N, 1)
    dg1 = lax.slice(deg_parts, (1, 0, 0), (2, N, 1)).reshape(N, 1)

    y1, dis = _tc1(x, W_in, b_in.reshape(1, D), W1, dg0, dg1)

    p1 = _edge_kernel(y1, idx4, w4)
    y2 = _tc2(p1[0], p1[1], y1, dis, b1.reshape(1, D), W2)

    p2 = _edge_kernel(y2, idx4, w4)
    logits = _tc3(p2[0], p2[1], y2, dis, b2.reshape(1, D),
                  batch.astype(jnp.int32).reshape(N, 1), Wc, bc.reshape(1, NCLS))
    return logits


# speedup vs baseline: 1.7672x; 1.7672x over previous
import functools

import jax
import jax.numpy as jnp
from jax import lax
from jax.experimental import pallas as pl
from jax.experimental.pallas import tpu as pltpu
from jax.experimental.pallas import tpu_sc as plsc

N = 10000
E = 320000
D = 128
NG = 64
NCLS = 10

NCORES = 2
NSUB = 16
NW = NCORES * NSUB
EPT = E // NW
CH = 80
CPT = 125
EPTP = CH * CPT
DUMP = 16
NDCH = N // DUMP
DCPT = -(-NDCH // NSUB)

def _zero_acc(page_v, acc_sh, sid):
    zero16 = jnp.zeros((16,), jnp.float32)

    @pl.loop(0, DUMP)
    def _(i):
        for j in range(D // 16):
            page_v[i, pl.ds(j * 16, 16)] = zero16

    @pl.loop(0, DCPT)
    def _(k):
        j = k * NSUB + sid

        @pl.when(j < NDCH)
        def _():
            pltpu.sync_copy(page_v, acc_sh.at[pl.ds(j * DUMP, DUMP)])

    plsc.subcore_barrier()


def _dump_acc(page_v, acc_sh, out_hbm, cid, sid):
    plsc.subcore_barrier()

    @pl.loop(0, DCPT)
    def _(k):
        j = k * NSUB + sid

        @pl.when(j < NDCH)
        def _():
            pltpu.sync_copy(acc_sh.at[pl.ds(j * DUMP, DUMP)], page_v)
            pltpu.sync_copy(page_v, out_hbm.at[cid, pl.ds(j * DUMP, DUMP)])


def _edge_body(y_hbm, idx_hbm, w_hbm, out_hbm,
               idx_v, wc_v, rows_v, acc_sh, sem):
    cid = lax.axis_index("c")
    sid = lax.axis_index("s")
    wid = sid * NCORES + cid

    _zero_acc(rows_v.at[pl.ds(0, DUMP)], acc_sh, sid)

    gdn = lax.GatherDimensionNumbers(
        offset_dims=(), collapsed_slice_dims=(0,), start_index_map=(0,))

    @pl.loop(0, CPT)
    def _(c):
        pltpu.sync_copy(idx_hbm.at[wid, c], idx_v)
        pltpu.sync_copy(w_hbm.at[wid, c], wc_v)
        pltpu.async_copy(y_hbm.at[idx_v.at[0]], rows_v, sem).wait()

        for g in range(CH // 16):
            wv = wc_v[0, pl.ds(g * 16, 16)]

            @pl.loop(0, 16, unroll=4)
            def _(r2):
                idxv = jnp.full((16, 1), r2, jnp.int32)
                wbc = lax.gather(wv, idxv, gdn, (1,),
                                 mode=lax.GatherScatterMode.PROMISE_IN_BOUNDS)
                rr = g * 16 + r2
                for j in range(D // 16):
                    rows_v[rr, pl.ds(j * 16, 16)] = (
                        rows_v[rr, pl.ds(j * 16, 16)] * wbc)

        pltpu.sync_copy(rows_v, acc_sh.at[idx_v.at[1]], add=True)

    _dump_acc(rows_v.at[pl.ds(0, DUMP)], acc_sh, out_hbm, cid, sid)


@functools.lru_cache(maxsize=None)
def _sc_kernels():
    mesh = plsc.VectorSubcoreMesh(core_axis_name="c", subcore_axis_name="s")
    params = pltpu.CompilerParams(needs_layout_passes=False)
    edge = pl.kernel(
        _edge_body,
        out_type=jax.ShapeDtypeStruct((NCORES, N, D), jnp.float32),
        mesh=mesh,
        compiler_params=params,
        scratch_types=[
            pltpu.VMEM((2, CH), jnp.int32),
            pltpu.VMEM((1, CH), jnp.float32),
            pltpu.VMEM((CH, D), jnp.float32),
            pltpu.VMEM_SHARED((N, D), jnp.float32),
            pltpu.SemaphoreType.DMA,
        ],
    )
    return (edge,)


def _edge_kernel(y, idx4, w4):
    return _sc_kernels()[0](y, idx4, w4)


RB = 1000
GRID = N // RB


def _tc1_body(x_ref, win_ref, bin_ref, w1_ref, dg0_ref, dg1_ref, y_ref, dis_ref):
    deg = dg0_ref[...] + dg1_ref[...] + 1.0
    dis = lax.rsqrt(deg)
    dis_ref[...] = dis
    h = jnp.maximum(jnp.dot(x_ref[...], win_ref[...],
                            preferred_element_type=jnp.float32) + bin_ref[...], 0.0)
    y_ref[...] = dis * jnp.dot(h, w1_ref[...], preferred_element_type=jnp.float32)


def _tc1(x, W_in, b_in, W1, dg0, dg1):
    return pl.pallas_call(
        _tc1_body,
        grid=(GRID,),
        in_specs=[
            pl.BlockSpec((RB, D), lambda i: (i, 0)),
            pl.BlockSpec((D, D), lambda i: (0, 0)),
            pl.BlockSpec((1, D), lambda i: (0, 0)),
            pl.BlockSpec((D, D), lambda i: (0, 0)),
            pl.BlockSpec((RB, 1), lambda i: (i, 0)),
            pl.BlockSpec((RB, 1), lambda i: (i, 0)),
        ],
        out_specs=[
            pl.BlockSpec((RB, D), lambda i: (i, 0)),
            pl.BlockSpec((RB, 1), lambda i: (i, 0)),
        ],
        out_shape=[
            jax.ShapeDtypeStruct((N, D), jnp.float32),
            jax.ShapeDtypeStruct((N, 1), jnp.float32),
        ],
    )(x, W_in, b_in, W1, dg0, dg1)


def _tc2_body(p0_ref, p1_ref, y_ref, dis_ref, b_ref, w_ref, out_ref):
    dis = dis_ref[...]
    h = jnp.maximum(dis * (p0_ref[...] + p1_ref[...] + y_ref[...]) + b_ref[...], 0.0)
    out_ref[...] = dis * jnp.dot(h, w_ref[...], preferred_element_type=jnp.float32)


def _tc2(p0, p1, y, dis, b, W):
    return pl.pallas_call(
        _tc2_body,
        grid=(GRID,),
        in_specs=[
            pl.BlockSpec((RB, D), lambda i: (i, 0)),
            pl.BlockSpec((RB, D), lambda i: (i, 0)),
            pl.BlockSpec((RB, D), lambda i: (i, 0)),
            pl.BlockSpec((RB, 1), lambda i: (i, 0)),
            pl.BlockSpec((1, D), lambda i: (0, 0)),
            pl.BlockSpec((D, D), lambda i: (0, 0)),
        ],
        out_specs=pl.BlockSpec((RB, D), lambda i: (i, 0)),
        out_shape=jax.ShapeDtypeStruct((N, D), jnp.float32),
    )(p0, p1, y, dis, b, W)


def _tc3_body(p0_ref, p1_ref, y_ref, dis_ref, b_ref, batch_ref, wc_ref, bc_ref,
              out_ref, sums_ref, cnts_ref):
    i = pl.program_id(0)

    @pl.when(i == 0)
    def _():
        sums_ref[...] = jnp.zeros_like(sums_ref)
        cnts_ref[...] = jnp.zeros_like(cnts_ref)

    dis = dis_ref[...]
    h = jnp.maximum(dis * (p0_ref[...] + p1_ref[...] + y_ref[...]) + b_ref[...], 0.0)
    b = batch_ref[...]
    iota = lax.broadcasted_iota(jnp.int32, (RB, NG), 1)
    onehot = (iota == b).astype(jnp.float32)
    dn = (((0,), (0,)), ((), ()))
    sums_ref[...] += lax.dot_general(onehot, h, dn,
                                     preferred_element_type=jnp.float32)
    cnts_ref[...] += lax.dot_general(onehot, jnp.ones((RB, 1), jnp.float32), dn,
                                     preferred_element_type=jnp.float32)

    @pl.when(i == GRID - 1)
    def _():
        rep = sums_ref[...] / jnp.maximum(cnts_ref[...], 1.0)
        out_ref[...] = jnp.dot(rep, wc_ref[...],
                               preferred_element_type=jnp.float32) + bc_ref[...]


def _tc3(p0, p1, y, dis, b, batch2, Wc, bc):
    return pl.pallas_call(
        _tc3_body,
        grid=(GRID,),
        in_specs=[
            pl.BlockSpec((RB, D), lambda i: (i, 0)),
            pl.BlockSpec((RB, D), lambda i: (i, 0)),
            pl.BlockSpec((RB, D), lambda i: (i, 0)),
            pl.BlockSpec((RB, 1), lambda i: (i, 0)),
            pl.BlockSpec((1, D), lambda i: (0, 0)),
            pl.BlockSpec((RB, 1), lambda i: (i, 0)),
            pl.BlockSpec((D, NCLS), lambda i: (0, 0)),
            pl.BlockSpec((1, NCLS), lambda i: (0, 0)),
        ],
        out_specs=pl.BlockSpec((NG, NCLS), lambda i: (0, 0)),
        out_shape=jax.ShapeDtypeStruct((NG, NCLS), jnp.float32),
        scratch_shapes=[
            pltpu.VMEM((NG, D), jnp.float32),
            pltpu.VMEM((NG, 1), jnp.float32),
        ],
    )(p0, p1, y, dis, b, batch2, Wc, bc)


def kernel(x, edge_index, edge_weights, batch, W_in, b_in, W1, b1, W2, b2, Wc, bc):
    src = edge_index[0].astype(jnp.int32)
    dst = edge_index[1].astype(jnp.int32)
    pad = ((0, 0), (0, EPTP - EPT))
    src3 = jnp.pad(src.reshape(NW, EPT), pad).reshape(NW, CPT, CH)
    dst3 = jnp.pad(dst.reshape(NW, EPT), pad).reshape(NW, CPT, CH)
    idx4 = jnp.stack([src3, dst3], axis=2)
    w4 = jnp.pad(edge_weights.astype(jnp.float32).reshape(NW, EPT),
                 pad).reshape(NW, CPT, 1, CH)

    ones_t = jnp.ones((N, D), jnp.float32)
    deg_parts = _edge_kernel(ones_t, idx4, w4)
    dg0 = lax.slice(deg_parts, (0, 0, 0), (1, N, 1)).reshape(N, 1)
    dg1 = lax.slice(deg_parts, (1, 0, 0), (2, N, 1)).reshape(N, 1)

    y1, dis = _tc1(x, W_in, b_in.reshape(1, D), W1, dg0, dg1)

    p1 = _edge_kernel(y1, idx4, w4)
    y2 = _tc2(p1[0], p1[1], y1, dis, b1.reshape(1, D), W2)

    p2 = _edge_kernel(y2, idx4, w4)
    logits = _tc3(p2[0], p2[1], y2, dis, b2.reshape(1, D),
                  batch.astype(jnp.int32).reshape(N, 1), Wc, bc.reshape(1, NCLS))
    return logits


# speedup vs baseline: 1.9845x; 1.1230x over previous
import functools

import jax
import jax.numpy as jnp
from jax import lax
from jax.experimental import pallas as pl
from jax.experimental.pallas import tpu as pltpu
from jax.experimental.pallas import tpu_sc as plsc

N = 10000
E = 320000
D = 128
NG = 64
NCLS = 10

NCORES = 2
NSUB = 16
NW = NCORES * NSUB
EPT = E // NW
CH = 80
CPT = 125
EPTP = CH * CPT
DUMP = 16
NDCH = N // DUMP
DCPT = -(-NDCH // NSUB)

def _zero_acc(page_v, acc_sh, sid):
    zero16 = jnp.zeros((16,), jnp.float32)

    @pl.loop(0, DUMP)
    def _(i):
        for j in range(D // 16):
            page_v[i, pl.ds(j * 16, 16)] = zero16

    @pl.loop(0, DCPT)
    def _(k):
        j = k * NSUB + sid

        @pl.when(j < NDCH)
        def _():
            pltpu.sync_copy(page_v, acc_sh.at[pl.ds(j * DUMP, DUMP)])

    plsc.subcore_barrier()


def _dump_acc(page_v, acc_sh, out_hbm, cid, sid):
    plsc.subcore_barrier()

    @pl.loop(0, DCPT)
    def _(k):
        j = k * NSUB + sid

        @pl.when(j < NDCH)
        def _():
            pltpu.sync_copy(acc_sh.at[pl.ds(j * DUMP, DUMP)], page_v)
            pltpu.sync_copy(page_v, out_hbm.at[cid, pl.ds(j * DUMP, DUMP)])


def _edge_body(y_hbm, idx_hbm, w_hbm, out_hbm,
               idx_v, wc_v, rows_v, acc_sh, sem):
    cid = lax.axis_index("c")
    sid = lax.axis_index("s")
    wid = sid * NCORES + cid

    _zero_acc(rows_v.at[pl.ds(0, DUMP)], acc_sh, sid)

    gdn = lax.GatherDimensionNumbers(
        offset_dims=(), collapsed_slice_dims=(0,), start_index_map=(0,))

    @pl.loop(0, CPT)
    def _(c):
        pltpu.sync_copy(idx_hbm.at[wid, c], idx_v)
        pltpu.sync_copy(w_hbm.at[wid, c], wc_v)
        pltpu.async_copy(y_hbm.at[idx_v.at[0]], rows_v, sem).wait()

        for g in range(CH // 16):
            wv = wc_v[0, pl.ds(g * 16, 16)]

            @pl.loop(0, 16, unroll=4)
            def _(r2):
                idxv = jnp.full((16, 1), r2, jnp.int32)
                wbc = lax.gather(wv, idxv, gdn, (1,),
                                 mode=lax.GatherScatterMode.PROMISE_IN_BOUNDS)
                rr = g * 16 + r2
                for j in range(D // 16):
                    rows_v[rr, pl.ds(j * 16, 16)] = (
                        rows_v[rr, pl.ds(j * 16, 16)] * wbc)

        pltpu.sync_copy(rows_v, acc_sh.at[idx_v.at[1]], add=True)

    _dump_acc(rows_v.at[pl.ds(0, DUMP)], acc_sh, out_hbm, cid, sid)


def _deg_body(idx_hbm, w_hbm, out_hbm, idx_v, wc_v, rows_v, acc_sh):
    cid = lax.axis_index("c")
    sid = lax.axis_index("s")
    wid = sid * NCORES + cid

    _zero_acc(rows_v.at[pl.ds(0, DUMP)], acc_sh, sid)

    gdn = lax.GatherDimensionNumbers(
        offset_dims=(), collapsed_slice_dims=(0,), start_index_map=(0,))

    @pl.loop(0, CPT)
    def _(c):
        pltpu.sync_copy(idx_hbm.at[wid, c], idx_v)
        pltpu.sync_copy(w_hbm.at[wid, c], wc_v)

        for g in range(CH // 16):
            wv = wc_v[0, pl.ds(g * 16, 16)]

            @pl.loop(0, 16, unroll=4)
            def _(r2):
                idxv = jnp.full((16, 1), r2, jnp.int32)
                wbc = lax.gather(wv, idxv, gdn, (1,),
                                 mode=lax.GatherScatterMode.PROMISE_IN_BOUNDS)
                rr = g * 16 + r2
                for j in range(D // 16):
                    rows_v[rr, pl.ds(j * 16, 16)] = wbc

        pltpu.sync_copy(rows_v, acc_sh.at[idx_v.at[1]], add=True)

    _dump_acc(rows_v.at[pl.ds(0, DUMP)], acc_sh, out_hbm, cid, sid)


@functools.lru_cache(maxsize=None)
def _sc_kernels():
    mesh = plsc.VectorSubcoreMesh(core_axis_name="c", subcore_axis_name="s")
    params = pltpu.CompilerParams(needs_layout_passes=False)
    edge = pl.kernel(
        _edge_body,
        out_type=jax.ShapeDtypeStruct((NCORES, N, D), jnp.float32),
        mesh=mesh,
        compiler_params=params,
        scratch_types=[
            pltpu.VMEM((2, CH), jnp.int32),
            pltpu.VMEM((1, CH), jnp.float32),
            pltpu.VMEM((CH, D), jnp.float32),
            pltpu.VMEM_SHARED((N, D), jnp.float32),
            pltpu.SemaphoreType.DMA,
        ],
    )
    deg = pl.kernel(
        _deg_body,
        out_type=jax.ShapeDtypeStruct((NCORES, N, D), jnp.float32),
        mesh=mesh,
        compiler_params=params,
        scratch_types=[
            pltpu.VMEM((2, CH), jnp.int32),
            pltpu.VMEM((1, CH), jnp.float32),
            pltpu.VMEM((CH, D), jnp.float32),
            pltpu.VMEM_SHARED((N, D), jnp.float32),
        ],
    )
    return (edge, deg)


def _edge_kernel(y, idx4, w4):
    return _sc_kernels()[0](y, idx4, w4)


def _deg_kernel(idx4, w4):
    return _sc_kernels()[1](idx4, w4)


RB = 1000
GRID = N // RB


def _tc1_body(x_ref, win_ref, bin_ref, w1_ref, dg0_ref, dg1_ref, y_ref, dis_ref):
    deg = dg0_ref[...] + dg1_ref[...] + 1.0
    dis = lax.rsqrt(deg)
    dis_ref[...] = dis
    h = jnp.maximum(jnp.dot(x_ref[...], win_ref[...],
                            preferred_element_type=jnp.float32) + bin_ref[...], 0.0)
    y_ref[...] = dis * jnp.dot(h, w1_ref[...], preferred_element_type=jnp.float32)


def _tc1(x, W_in, b_in, W1, dg0, dg1):
    return pl.pallas_call(
        _tc1_body,
        grid=(GRID,),
        in_specs=[
            pl.BlockSpec((RB, D), lambda i: (i, 0)),
            pl.BlockSpec((D, D), lambda i: (0, 0)),
            pl.BlockSpec((1, D), lambda i: (0, 0)),
            pl.BlockSpec((D, D), lambda i: (0, 0)),
            pl.BlockSpec((RB, 1), lambda i: (i, 0)),
            pl.BlockSpec((RB, 1), lambda i: (i, 0)),
        ],
        out_specs=[
            pl.BlockSpec((RB, D), lambda i: (i, 0)),
            pl.BlockSpec((RB, 1), lambda i: (i, 0)),
        ],
        out_shape=[
            jax.ShapeDtypeStruct((N, D), jnp.float32),
            jax.ShapeDtypeStruct((N, 1), jnp.float32),
        ],
    )(x, W_in, b_in, W1, dg0, dg1)


def _tc2_body(p0_ref, p1_ref, y_ref, dis_ref, b_ref, w_ref, out_ref):
    dis = dis_ref[...]
    h = jnp.maximum(dis * (p0_ref[...] + p1_ref[...] + y_ref[...]) + b_ref[...], 0.0)
    out_ref[...] = dis * jnp.dot(h, w_ref[...], preferred_element_type=jnp.float32)


def _tc2(p0, p1, y, dis, b, W):
    return pl.pallas_call(
        _tc2_body,
        grid=(GRID,),
        in_specs=[
            pl.BlockSpec((RB, D), lambda i: (i, 0)),
            pl.BlockSpec((RB, D), lambda i: (i, 0)),
            pl.BlockSpec((RB, D), lambda i: (i, 0)),
            pl.BlockSpec((RB, 1), lambda i: (i, 0)),
            pl.BlockSpec((1, D), lambda i: (0, 0)),
            pl.BlockSpec((D, D), lambda i: (0, 0)),
        ],
        out_specs=pl.BlockSpec((RB, D), lambda i: (i, 0)),
        out_shape=jax.ShapeDtypeStruct((N, D), jnp.float32),
    )(p0, p1, y, dis, b, W)


def _tc3_body(p0_ref, p1_ref, y_ref, dis_ref, b_ref, batch_ref, wc_ref, bc_ref,
              out_ref, sums_ref, cnts_ref):
    i = pl.program_id(0)

    @pl.when(i == 0)
    def _():
        sums_ref[...] = jnp.zeros_like(sums_ref)
        cnts_ref[...] = jnp.zeros_like(cnts_ref)

    dis = dis_ref[...]
    h = jnp.maximum(dis * (p0_ref[...] + p1_ref[...] + y_ref[...]) + b_ref[...], 0.0)
    b = batch_ref[...]
    iota = lax.broadcasted_iota(jnp.int32, (RB, NG), 1)
    onehot = (iota == b).astype(jnp.float32)
    dn = (((0,), (0,)), ((), ()))
    sums_ref[...] += lax.dot_general(onehot, h, dn,
                                     preferred_element_type=jnp.float32)
    cnts_ref[...] += lax.dot_general(onehot, jnp.ones((RB, 1), jnp.float32), dn,
                                     preferred_element_type=jnp.float32)

    @pl.when(i == GRID - 1)
    def _():
        rep = sums_ref[...] / jnp.maximum(cnts_ref[...], 1.0)
        out_ref[...] = jnp.dot(rep, wc_ref[...],
                               preferred_element_type=jnp.float32) + bc_ref[...]


def _tc3(p0, p1, y, dis, b, batch2, Wc, bc):
    return pl.pallas_call(
        _tc3_body,
        grid=(GRID,),
        in_specs=[
            pl.BlockSpec((RB, D), lambda i: (i, 0)),
            pl.BlockSpec((RB, D), lambda i: (i, 0)),
            pl.BlockSpec((RB, D), lambda i: (i, 0)),
            pl.BlockSpec((RB, 1), lambda i: (i, 0)),
            pl.BlockSpec((1, D), lambda i: (0, 0)),
            pl.BlockSpec((RB, 1), lambda i: (i, 0)),
            pl.BlockSpec((D, NCLS), lambda i: (0, 0)),
            pl.BlockSpec((1, NCLS), lambda i: (0, 0)),
        ],
        out_specs=pl.BlockSpec((NG, NCLS), lambda i: (0, 0)),
        out_shape=jax.ShapeDtypeStruct((NG, NCLS), jnp.float32),
        scratch_shapes=[
            pltpu.VMEM((NG, D), jnp.float32),
            pltpu.VMEM((NG, 1), jnp.float32),
        ],
    )(p0, p1, y, dis, b, batch2, Wc, bc)


def kernel(x, edge_index, edge_weights, batch, W_in, b_in, W1, b1, W2, b2, Wc, bc):
    src = edge_index[0].astype(jnp.int32)
    dst = edge_index[1].astype(jnp.int32)
    pad = ((0, 0), (0, EPTP - EPT))
    src3 = jnp.pad(src.reshape(NW, EPT), pad).reshape(NW, CPT, CH)
    dst3 = jnp.pad(dst.reshape(NW, EPT), pad).reshape(NW, CPT, CH)
    idx4 = jnp.stack([src3, dst3], axis=2)
    w4 = jnp.pad(edge_weights.astype(jnp.float32).reshape(NW, EPT),
                 pad).reshape(NW, CPT, 1, CH)

    deg_parts = _deg_kernel(idx4, w4)
    dg0 = lax.slice(deg_parts, (0, 0, 0), (1, N, 1)).reshape(N, 1)
    dg1 = lax.slice(deg_parts, (1, 0, 0), (2, N, 1)).reshape(N, 1)

    y1, dis = _tc1(x, W_in, b_in.reshape(1, D), W1, dg0, dg1)

    p1 = _edge_kernel(y1, idx4, w4)
    y2 = _tc2(p1[0], p1[1], y1, dis, b1.reshape(1, D), W2)

    p2 = _edge_kernel(y2, idx4, w4)
    logits = _tc3(p2[0], p2[1], y2, dis, b2.reshape(1, D),
                  batch.astype(jnp.int32).reshape(N, 1), Wc, bc.reshape(1, NCLS))
    return logits


# speedup vs baseline: 2.1448x; 1.0808x over previous
import functools

import jax
import jax.numpy as jnp
from jax import lax
from jax.experimental import pallas as pl
from jax.experimental.pallas import tpu as pltpu
from jax.experimental.pallas import tpu_sc as plsc

N = 10000
E = 320000
D = 128
NG = 64
NCLS = 10

NCORES = 2
NSUB = 16
NW = NCORES * NSUB
EPT = E // NW
CH = 80
CPT = 126
EPTP = CH * CPT
DUMP = 16
NDCH = N // DUMP
DCPT = -(-NDCH // NSUB)

def _zero_acc(page_v, acc_sh, sid):
    zero16 = jnp.zeros((16,), jnp.float32)

    @pl.loop(0, DUMP)
    def _(i):
        for j in range(D // 16):
            page_v[i, pl.ds(j * 16, 16)] = zero16

    @pl.loop(0, DCPT)
    def _(k):
        j = k * NSUB + sid

        @pl.when(j < NDCH)
        def _():
            pltpu.sync_copy(page_v, acc_sh.at[pl.ds(j * DUMP, DUMP)])

    plsc.subcore_barrier()


def _dump_acc(page_v, acc_sh, out_hbm, cid, sid):
    plsc.subcore_barrier()

    @pl.loop(0, DCPT)
    def _(k):
        j = k * NSUB + sid

        @pl.when(j < NDCH)
        def _():
            pltpu.sync_copy(acc_sh.at[pl.ds(j * DUMP, DUMP)], page_v)
            pltpu.sync_copy(page_v, out_hbm.at[cid, pl.ds(j * DUMP, DUMP)])


def _edge_body(y_hbm, idx_hbm, w_hbm, out_hbm,
               idx0_v, idx1_v, wc0_v, wc1_v, rows0_v, rows1_v, acc_sh,
               sem0, sem1):
    cid = lax.axis_index("c")
    sid = lax.axis_index("s")
    wid = sid * NCORES + cid
    idxs = (idx0_v, idx1_v)
    wcs = (wc0_v, wc1_v)
    rows = (rows0_v, rows1_v)
    sems = (sem0, sem1)

    _zero_acc(rows0_v.at[pl.ds(0, DUMP)], acc_sh, sid)

    gdn = lax.GatherDimensionNumbers(
        offset_dims=(), collapsed_slice_dims=(0,), start_index_map=(0,))

    for b in range(2):
        pltpu.sync_copy(idx_hbm.at[wid, b], idxs[b])
        pltpu.sync_copy(w_hbm.at[wid, b], wcs[b])
        pltpu.async_copy(y_hbm.at[idxs[b].at[0]], rows[b], sems[b])

    @pl.loop(0, CPT, step=2)
    def _(c):
        for b in range(2):
            pltpu.make_async_copy(y_hbm.at[idxs[b].at[0]], rows[b],
                                  sems[b]).wait()

            for g in range(CH // 16):
                wv = wcs[b][0, pl.ds(g * 16, 16)]

                @pl.loop(0, 16, unroll=4)
                def _(r2):
                    idxv = jnp.full((16, 1), r2, jnp.int32)
                    wbc = lax.gather(wv, idxv, gdn, (1,),
                                     mode=lax.GatherScatterMode.PROMISE_IN_BOUNDS)
                    rr = g * 16 + r2
                    for j in range(D // 16):
                        rows[b][rr, pl.ds(j * 16, 16)] = (
                            rows[b][rr, pl.ds(j * 16, 16)] * wbc)

            pltpu.sync_copy(rows[b], acc_sh.at[idxs[b].at[1]], add=True)

            @pl.when(c + (b + 2) < CPT)
            def _():
                pltpu.sync_copy(idx_hbm.at[wid, c + (b + 2)], idxs[b])
                pltpu.sync_copy(w_hbm.at[wid, c + (b + 2)], wcs[b])
                pltpu.async_copy(y_hbm.at[idxs[b].at[0]], rows[b], sems[b])

    _dump_acc(rows0_v.at[pl.ds(0, DUMP)], acc_sh, out_hbm, cid, sid)


def _deg_body(idx_hbm, w_hbm, out_hbm, idx_v, wc_v, rows_v, acc_sh):
    cid = lax.axis_index("c")
    sid = lax.axis_index("s")
    wid = sid * NCORES + cid

    _zero_acc(rows_v.at[pl.ds(0, DUMP)], acc_sh, sid)

    gdn = lax.GatherDimensionNumbers(
        offset_dims=(), collapsed_slice_dims=(0,), start_index_map=(0,))

    @pl.loop(0, CPT)
    def _(c):
        pltpu.sync_copy(idx_hbm.at[wid, c], idx_v)
        pltpu.sync_copy(w_hbm.at[wid, c], wc_v)

        for g in range(CH // 16):
            wv = wc_v[0, pl.ds(g * 16, 16)]

            @pl.loop(0, 16, unroll=4)
            def _(r2):
                idxv = jnp.full((16, 1), r2, jnp.int32)
                wbc = lax.gather(wv, idxv, gdn, (1,),
                                 mode=lax.GatherScatterMode.PROMISE_IN_BOUNDS)
                rr = g * 16 + r2
                for j in range(D // 16):
                    rows_v[rr, pl.ds(j * 16, 16)] = wbc

        pltpu.sync_copy(rows_v, acc_sh.at[idx_v.at[1]], add=True)

    _dump_acc(rows_v.at[pl.ds(0, DUMP)], acc_sh, out_hbm, cid, sid)


@functools.lru_cache(maxsize=None)
def _sc_kernels():
    mesh = plsc.VectorSubcoreMesh(core_axis_name="c", subcore_axis_name="s")
    params = pltpu.CompilerParams(needs_layout_passes=False)
    edge = pl.kernel(
        _edge_body,
        out_type=jax.ShapeDtypeStruct((NCORES, N, D), jnp.float32),
        mesh=mesh,
        compiler_params=params,
        scratch_types=[
            pltpu.VMEM((2, CH), jnp.int32),
            pltpu.VMEM((2, CH), jnp.int32),
            pltpu.VMEM((1, CH), jnp.float32),
            pltpu.VMEM((1, CH), jnp.float32),
            pltpu.VMEM((CH, D), jnp.float32),
            pltpu.VMEM((CH, D), jnp.float32),
            pltpu.VMEM_SHARED((N, D), jnp.float32),
            pltpu.SemaphoreType.DMA,
            pltpu.SemaphoreType.DMA,
        ],
    )
    deg = pl.kernel(
        _deg_body,
        out_type=jax.ShapeDtypeStruct((NCORES, N, D), jnp.float32),
        mesh=mesh,
        compiler_params=params,
        scratch_types=[
            pltpu.VMEM((2, CH), jnp.int32),
            pltpu.VMEM((1, CH), jnp.float32),
            pltpu.VMEM((CH, D), jnp.float32),
            pltpu.VMEM_SHARED((N, D), jnp.float32),
        ],
    )
    return (edge, deg)


def _edge_kernel(y, idx4, w4):
    return _sc_kernels()[0](y, idx4, w4)


def _deg_kernel(idx4, w4):
    return _sc_kernels()[1](idx4, w4)


RB = 1000
GRID = N // RB


def _tc1_body(x_ref, win_ref, bin_ref, w1_ref, dg0_ref, dg1_ref, y_ref, dis_ref):
    deg = dg0_ref[...] + dg1_ref[...] + 1.0
    dis = lax.rsqrt(deg)
    dis_ref[...] = dis
    h = jnp.maximum(jnp.dot(x_ref[...], win_ref[...],
                            preferred_element_type=jnp.float32) + bin_ref[...], 0.0)
    y_ref[...] = dis * jnp.dot(h, w1_ref[...], preferred_element_type=jnp.float32)


def _tc1(x, W_in, b_in, W1, dg0, dg1):
    return pl.pallas_call(
        _tc1_body,
        grid=(GRID,),
        in_specs=[
            pl.BlockSpec((RB, D), lambda i: (i, 0)),
            pl.BlockSpec((D, D), lambda i: (0, 0)),
            pl.BlockSpec((1, D), lambda i: (0, 0)),
            pl.BlockSpec((D, D), lambda i: (0, 0)),
            pl.BlockSpec((RB, 1), lambda i: (i, 0)),
            pl.BlockSpec((RB, 1), lambda i: (i, 0)),
        ],
        out_specs=[
            pl.BlockSpec((RB, D), lambda i: (i, 0)),
            pl.BlockSpec((RB, 1), lambda i: (i, 0)),
        ],
        out_shape=[
            jax.ShapeDtypeStruct((N, D), jnp.float32),
            jax.ShapeDtypeStruct((N, 1), jnp.float32),
        ],
    )(x, W_in, b_in, W1, dg0, dg1)


def _tc2_body(p0_ref, p1_ref, y_ref, dis_ref, b_ref, w_ref, out_ref):
    dis = dis_ref[...]
    h = jnp.maximum(dis * (p0_ref[...] + p1_ref[...] + y_ref[...]) + b_ref[...], 0.0)
    out_ref[...] = dis * jnp.dot(h, w_ref[...], preferred_element_type=jnp.float32)


def _tc2(p0, p1, y, dis, b, W):
    return pl.pallas_call(
        _tc2_body,
        grid=(GRID,),
        in_specs=[
            pl.BlockSpec((RB, D), lambda i: (i, 0)),
            pl.BlockSpec((RB, D), lambda i: (i, 0)),
            pl.BlockSpec((RB, D), lambda i: (i, 0)),
            pl.BlockSpec((RB, 1), lambda i: (i, 0)),
            pl.BlockSpec((1, D), lambda i: (0, 0)),
            pl.BlockSpec((D, D), lambda i: (0, 0)),
        ],
        out_specs=pl.BlockSpec((RB, D), lambda i: (i, 0)),
        out_shape=jax.ShapeDtypeStruct((N, D), jnp.float32),
    )(p0, p1, y, dis, b, W)


def _tc3_body(p0_ref, p1_ref, y_ref, dis_ref, b_ref, batch_ref, wc_ref, bc_ref,
              out_ref, sums_ref, cnts_ref):
    i = pl.program_id(0)

    @pl.when(i == 0)
    def _():
        sums_ref[...] = jnp.zeros_like(sums_ref)
        cnts_ref[...] = jnp.zeros_like(cnts_ref)

    dis = dis_ref[...]
    h = jnp.maximum(dis * (p0_ref[...] + p1_ref[...] + y_ref[...]) + b_ref[...], 0.0)
    b = batch_ref[...]
    iota = lax.broadcasted_iota(jnp.int32, (RB, NG), 1)
    onehot = (iota == b).astype(jnp.float32)
    dn = (((0,), (0,)), ((), ()))
    sums_ref[...] += lax.dot_general(onehot, h, dn,
                                     preferred_element_type=jnp.float32)
    cnts_ref[...] += lax.dot_general(onehot, jnp.ones((RB, 1), jnp.float32), dn,
                                     preferred_element_type=jnp.float32)

    @pl.when(i == GRID - 1)
    def _():
        rep = sums_ref[...] / jnp.maximum(cnts_ref[...], 1.0)
        out_ref[...] = jnp.dot(rep, wc_ref[...],
                               preferred_element_type=jnp.float32) + bc_ref[...]


def _tc3(p0, p1, y, dis, b, batch2, Wc, bc):
    return pl.pallas_call(
        _tc3_body,
        grid=(GRID,),
        in_specs=[
            pl.BlockSpec((RB, D), lambda i: (i, 0)),
            pl.BlockSpec((RB, D), lambda i: (i, 0)),
            pl.BlockSpec((RB, D), lambda i: (i, 0)),
            pl.BlockSpec((RB, 1), lambda i: (i, 0)),
            pl.BlockSpec((1, D), lambda i: (0, 0)),
            pl.BlockSpec((RB, 1), lambda i: (i, 0)),
            pl.BlockSpec((D, NCLS), lambda i: (0, 0)),
            pl.BlockSpec((1, NCLS), lambda i: (0, 0)),
        ],
        out_specs=pl.BlockSpec((NG, NCLS), lambda i: (0, 0)),
        out_shape=jax.ShapeDtypeStruct((NG, NCLS), jnp.float32),
        scratch_shapes=[
            pltpu.VMEM((NG, D), jnp.float32),
            pltpu.VMEM((NG, 1), jnp.float32),
        ],
    )(p0, p1, y, dis, b, batch2, Wc, bc)


def kernel(x, edge_index, edge_weights, batch, W_in, b_in, W1, b1, W2, b2, Wc, bc):
    src = edge_index[0].astype(jnp.int32)
    dst = edge_index[1].astype(jnp.int32)
    pad = ((0, 0), (0, EPTP - EPT))
    src3 = jnp.pad(src.reshape(NW, EPT), pad).reshape(NW, CPT, CH)
    dst3 = jnp.pad(dst.reshape(NW, EPT), pad).reshape(NW, CPT, CH)
    idx4 = jnp.stack([src3, dst3], axis=2)
    w4 = jnp.pad(edge_weights.astype(jnp.float32).reshape(NW, EPT),
                 pad).reshape(NW, CPT, 1, CH)

    deg_parts = _deg_kernel(idx4, w4)
    dg0 = lax.slice(deg_parts, (0, 0, 0), (1, N, 1)).reshape(N, 1)
    dg1 = lax.slice(deg_parts, (1, 0, 0), (2, N, 1)).reshape(N, 1)

    y1, dis = _tc1(x, W_in, b_in.reshape(1, D), W1, dg0, dg1)

    p1 = _edge_kernel(y1, idx4, w4)
    y2 = _tc2(p1[0], p1[1], y1, dis, b1.reshape(1, D), W2)

    p2 = _edge_kernel(y2, idx4, w4)
    logits = _tc3(p2[0], p2[1], y2, dis, b2.reshape(1, D),
                  batch.astype(jnp.int32).reshape(N, 1), Wc, bc.reshape(1, NCLS))
    return logits


# speedup vs baseline: 2.2265x; 1.0381x over previous
import functools

import jax
import jax.numpy as jnp
from jax import lax
from jax.experimental import pallas as pl
from jax.experimental.pallas import tpu as pltpu
from jax.experimental.pallas import tpu_sc as plsc

N = 10000
E = 320000
D = 128
NG = 64
NCLS = 10

NCORES = 2
NSUB = 16
NW = NCORES * NSUB
EPT = E // NW
CH = 80
CPT = 126
EPTP = CH * CPT
DUMP = 16
NDCH = N // DUMP
DCPT = -(-NDCH // NSUB)

def _zero_acc(page_v, acc_sh, sid):
    zero16 = jnp.zeros((16,), jnp.float32)

    @pl.loop(0, DUMP)
    def _(i):
        for j in range(D // 16):
            page_v[i, pl.ds(j * 16, 16)] = zero16

    @pl.loop(0, DCPT)
    def _(k):
        j = k * NSUB + sid

        @pl.when(j < NDCH)
        def _():
            pltpu.sync_copy(page_v, acc_sh.at[pl.ds(j * DUMP, DUMP)])

    plsc.subcore_barrier()


def _dump_acc(page_v, acc_sh, out_hbm, cid, sid):
    plsc.subcore_barrier()

    @pl.loop(0, DCPT)
    def _(k):
        j = k * NSUB + sid

        @pl.when(j < NDCH)
        def _():
            pltpu.sync_copy(acc_sh.at[pl.ds(j * DUMP, DUMP)], page_v)
            pltpu.sync_copy(page_v, out_hbm.at[cid, pl.ds(j * DUMP, DUMP)])


def _edge_body(y_hbm, idx_hbm, w_hbm, out_hbm,
               idx0_v, idx1_v, wc0_v, wc1_v, rows0_v, rows1_v, acc_sh,
               sem0, sem1):
    cid = lax.axis_index("c")
    sid = lax.axis_index("s")
    wid = sid * NCORES + cid
    idxs = (idx0_v, idx1_v)
    wcs = (wc0_v, wc1_v)
    rows = (rows0_v, rows1_v)
    sems = (sem0, sem1)

    _zero_acc(rows0_v.at[pl.ds(0, DUMP)], acc_sh, sid)

    gdn = lax.GatherDimensionNumbers(
        offset_dims=(), collapsed_slice_dims=(0,), start_index_map=(0,))

    for b in range(2):
        pltpu.sync_copy(idx_hbm.at[wid, b], idxs[b])
        pltpu.sync_copy(w_hbm.at[wid, b], wcs[b])
        pltpu.async_copy(y_hbm.at[idxs[b].at[0]], rows[b], sems[b])

    @pl.loop(0, CPT, step=2)
    def _(c):
        for b in range(2):
            pltpu.make_async_copy(y_hbm.at[idxs[b].at[0]], rows[b],
                                  sems[b]).wait()

            for g in range(CH // 16):
                wv = wcs[b][0, pl.ds(g * 16, 16)]

                @pl.loop(0, 16, unroll=4)
                def _(r2):
                    idxv = jnp.full((16, 1), r2, jnp.int32)
                    wbc = lax.gather(wv, idxv, gdn, (1,),
                                     mode=lax.GatherScatterMode.PROMISE_IN_BOUNDS)
                    rr = g * 16 + r2
                    for j in range(D // 16):
                        rows[b][rr, pl.ds(j * 16, 16)] = (
                            rows[b][rr, pl.ds(j * 16, 16)] * wbc)

            pltpu.sync_copy(rows[b], acc_sh.at[idxs[b].at[1]], add=True)

            @pl.when(c + (b + 2) < CPT)
            def _():
                pltpu.sync_copy(idx_hbm.at[wid, c + (b + 2)], idxs[b])
                pltpu.sync_copy(w_hbm.at[wid, c + (b + 2)], wcs[b])
                pltpu.async_copy(y_hbm.at[idxs[b].at[0]], rows[b], sems[b])

    _dump_acc(rows0_v.at[pl.ds(0, DUMP)], acc_sh, out_hbm, cid, sid)


def _deg_body(idx_hbm, w_hbm, out_hbm, idx_v, wc_v, rows_v, acc_sh):
    cid = lax.axis_index("c")
    sid = lax.axis_index("s")
    wid = sid * NCORES + cid

    _zero_acc(rows_v.at[pl.ds(0, DUMP)], acc_sh, sid)

    zero16 = jnp.zeros((16,), jnp.float32)

    @pl.loop(0, CH)
    def _(i):
        for j in range(D // 16):
            rows_v[i, pl.ds(j * 16, 16)] = zero16

    gdn = lax.GatherDimensionNumbers(
        offset_dims=(), collapsed_slice_dims=(0,), start_index_map=(0,))

    @pl.loop(0, CPT)
    def _(c):
        pltpu.sync_copy(idx_hbm.at[wid, c], idx_v)
        pltpu.sync_copy(w_hbm.at[wid, c], wc_v)

        for g in range(CH // 16):
            wv = wc_v[0, pl.ds(g * 16, 16)]

            @pl.loop(0, 16, unroll=4)
            def _(r2):
                idxv = jnp.full((16, 1), r2, jnp.int32)
                wbc = lax.gather(wv, idxv, gdn, (1,),
                                 mode=lax.GatherScatterMode.PROMISE_IN_BOUNDS)
                rows_v[g * 16 + r2, pl.ds(0, 16)] = wbc

        pltpu.sync_copy(rows_v, acc_sh.at[idx_v.at[1]], add=True)

    _dump_acc(rows_v.at[pl.ds(0, DUMP)], acc_sh, out_hbm, cid, sid)


@functools.lru_cache(maxsize=None)
def _sc_kernels():
    mesh = plsc.VectorSubcoreMesh(core_axis_name="c", subcore_axis_name="s")
    params = pltpu.CompilerParams(needs_layout_passes=False)
    edge = pl.kernel(
        _edge_body,
        out_type=jax.ShapeDtypeStruct((NCORES, N, D), jnp.float32),
        mesh=mesh,
        compiler_params=params,
        scratch_types=[
            pltpu.VMEM((2, CH), jnp.int32),
            pltpu.VMEM((2, CH), jnp.int32),
            pltpu.VMEM((1, CH), jnp.float32),
            pltpu.VMEM((1, CH), jnp.float32),
            pltpu.VMEM((CH, D), jnp.float32),
            pltpu.VMEM((CH, D), jnp.float32),
            pltpu.VMEM_SHARED((N, D), jnp.float32),
            pltpu.SemaphoreType.DMA,
            pltpu.SemaphoreType.DMA,
        ],
    )
    deg = pl.kernel(
        _deg_body,
        out_type=jax.ShapeDtypeStruct((NCORES, N, D), jnp.float32),
        mesh=mesh,
        compiler_params=params,
        scratch_types=[
            pltpu.VMEM((2, CH), jnp.int32),
            pltpu.VMEM((1, CH), jnp.float32),
            pltpu.VMEM((CH, D), jnp.float32),
            pltpu.VMEM_SHARED((N, D), jnp.float32),
        ],
    )
    return (edge, deg)


def _edge_kernel(y, idx4, w4):
    return _sc_kernels()[0](y, idx4, w4)


def _deg_kernel(idx4, w4):
    return _sc_kernels()[1](idx4, w4)


RB = 1000
GRID = N // RB


def _tc1_body(x_ref, win_ref, bin_ref, w1_ref, dg0_ref, dg1_ref, y_ref, dis_ref):
    deg = dg0_ref[...] + dg1_ref[...] + 1.0
    dis = lax.rsqrt(deg)
    dis_ref[...] = dis
    h = jnp.maximum(jnp.dot(x_ref[...], win_ref[...],
                            preferred_element_type=jnp.float32) + bin_ref[...], 0.0)
    y_ref[...] = dis * jnp.dot(h, w1_ref[...], preferred_element_type=jnp.float32)


def _tc1(x, W_in, b_in, W1, dg0, dg1):
    return pl.pallas_call(
        _tc1_body,
        grid=(GRID,),
        in_specs=[
            pl.BlockSpec((RB, D), lambda i: (i, 0)),
            pl.BlockSpec((D, D), lambda i: (0, 0)),
            pl.BlockSpec((1, D), lambda i: (0, 0)),
            pl.BlockSpec((D, D), lambda i: (0, 0)),
            pl.BlockSpec((RB, 1), lambda i: (i, 0)),
            pl.BlockSpec((RB, 1), lambda i: (i, 0)),
        ],
        out_specs=[
            pl.BlockSpec((RB, D), lambda i: (i, 0)),
            pl.BlockSpec((RB, 1), lambda i: (i, 0)),
        ],
        out_shape=[
            jax.ShapeDtypeStruct((N, D), jnp.float32),
            jax.ShapeDtypeStruct((N, 1), jnp.float32),
        ],
    )(x, W_in, b_in, W1, dg0, dg1)


def _tc2_body(p0_ref, p1_ref, y_ref, dis_ref, b_ref, w_ref, out_ref):
    dis = dis_ref[...]
    h = jnp.maximum(dis * (p0_ref[...] + p1_ref[...] + y_ref[...]) + b_ref[...], 0.0)
    out_ref[...] = dis * jnp.dot(h, w_ref[...], preferred_element_type=jnp.float32)


def _tc2(p0, p1, y, dis, b, W):
    return pl.pallas_call(
        _tc2_body,
        grid=(GRID,),
        in_specs=[
            pl.BlockSpec((RB, D), lambda i: (i, 0)),
            pl.BlockSpec((RB, D), lambda i: (i, 0)),
            pl.BlockSpec((RB, D), lambda i: (i, 0)),
            pl.BlockSpec((RB, 1), lambda i: (i, 0)),
            pl.BlockSpec((1, D), lambda i: (0, 0)),
            pl.BlockSpec((D, D), lambda i: (0, 0)),
        ],
        out_specs=pl.BlockSpec((RB, D), lambda i: (i, 0)),
        out_shape=jax.ShapeDtypeStruct((N, D), jnp.float32),
    )(p0, p1, y, dis, b, W)


def _tc3_body(p0_ref, p1_ref, y_ref, dis_ref, b_ref, batch_ref, wc_ref, bc_ref,
              out_ref, sums_ref, cnts_ref):
    i = pl.program_id(0)

    @pl.when(i == 0)
    def _():
        sums_ref[...] = jnp.zeros_like(sums_ref)
        cnts_ref[...] = jnp.zeros_like(cnts_ref)

    dis = dis_ref[...]
    h = jnp.maximum(dis * (p0_ref[...] + p1_ref[...] + y_ref[...]) + b_ref[...], 0.0)
    b = batch_ref[...]
    iota = lax.broadcasted_iota(jnp.int32, (RB, NG), 1)
    onehot = (iota == b).astype(jnp.float32)
    dn = (((0,), (0,)), ((), ()))
    sums_ref[...] += lax.dot_general(onehot, h, dn,
                                     preferred_element_type=jnp.float32)
    cnts_ref[...] += lax.dot_general(onehot, jnp.ones((RB, 1), jnp.float32), dn,
                                     preferred_element_type=jnp.float32)

    @pl.when(i == GRID - 1)
    def _():
        rep = sums_ref[...] / jnp.maximum(cnts_ref[...], 1.0)
        out_ref[...] = jnp.dot(rep, wc_ref[...],
                               preferred_element_type=jnp.float32) + bc_ref[...]


def _tc3(p0, p1, y, dis, b, batch2, Wc, bc):
    return pl.pallas_call(
        _tc3_body,
        grid=(GRID,),
        in_specs=[
            pl.BlockSpec((RB, D), lambda i: (i, 0)),
            pl.BlockSpec((RB, D), lambda i: (i, 0)),
            pl.BlockSpec((RB, D), lambda i: (i, 0)),
            pl.BlockSpec((RB, 1), lambda i: (i, 0)),
            pl.BlockSpec((1, D), lambda i: (0, 0)),
            pl.BlockSpec((RB, 1), lambda i: (i, 0)),
            pl.BlockSpec((D, NCLS), lambda i: (0, 0)),
            pl.BlockSpec((1, NCLS), lambda i: (0, 0)),
        ],
        out_specs=pl.BlockSpec((NG, NCLS), lambda i: (0, 0)),
        out_shape=jax.ShapeDtypeStruct((NG, NCLS), jnp.float32),
        scratch_shapes=[
            pltpu.VMEM((NG, D), jnp.float32),
            pltpu.VMEM((NG, 1), jnp.float32),
        ],
    )(p0, p1, y, dis, b, batch2, Wc, bc)


def kernel(x, edge_index, edge_weights, batch, W_in, b_in, W1, b1, W2, b2, Wc, bc):
    src = edge_index[0].astype(jnp.int32)
    dst = edge_index[1].astype(jnp.int32)
    pad = ((0, 0), (0, EPTP - EPT))
    src3 = jnp.pad(src.reshape(NW, EPT), pad).reshape(NW, CPT, CH)
    dst3 = jnp.pad(dst.reshape(NW, EPT), pad).reshape(NW, CPT, CH)
    idx4 = jnp.stack([src3, dst3], axis=2)
    w4 = jnp.pad(edge_weights.astype(jnp.float32).reshape(NW, EPT),
                 pad).reshape(NW, CPT, 1, CH)

    deg_parts = _deg_kernel(idx4, w4)
    dg0 = lax.slice(deg_parts, (0, 0, 0), (1, N, 1)).reshape(N, 1)
    dg1 = lax.slice(deg_parts, (1, 0, 0), (2, N, 1)).reshape(N, 1)

    y1, dis = _tc1(x, W_in, b_in.reshape(1, D), W1, dg0, dg1)

    p1 = _edge_kernel(y1, idx4, w4)
    y2 = _tc2(p1[0], p1[1], y1, dis, b1.reshape(1, D), W2)

    p2 = _edge_kernel(y2, idx4, w4)
    logits = _tc3(p2[0], p2[1], y2, dis, b2.reshape(1, D),
                  batch.astype(jnp.int32).reshape(N, 1), Wc, bc.reshape(1, NCLS))
    return logits
